# Initial kernel scaffold; baseline (speedup 1.0000x reference)
import jax
import jax.numpy as jnp
from jax.experimental import pallas as pl


def kernel(x, gate_w, w1, w2, b1, b2):
    raise NotImplementedError("write your pallas kernel here")


# speedup vs baseline: 8.7897x; 8.7897x over previous
import functools

import jax
import jax.numpy as jnp
from jax.experimental import pallas as pl
from jax.experimental.pallas import tpu as pltpu

D_MODEL_ = 1024
D_HID_ = 4096
E_ = 8
TOPK_ = 2

H_TILE = 1024


def _router_body(x_ref, gw_ref, wt_ref):
    logits = jnp.dot(x_ref[...], gw_ref[...], preferred_element_type=jnp.float32)
    m = jnp.max(logits, axis=-1, keepdims=True)
    ex = jnp.exp(logits - m)
    probs = ex / jnp.sum(ex, axis=-1, keepdims=True)

    ncols = probs.shape[-1]
    iota = jax.lax.broadcasted_iota(jnp.int32, probs.shape, 1)
    big = jnp.int32(ncols)

    m1 = jnp.max(probs, axis=-1, keepdims=True)
    i1 = jnp.min(jnp.where(probs == m1, iota, big), axis=-1, keepdims=True)
    mask1 = iota == i1
    probs2 = jnp.where(mask1, -jnp.inf, probs)
    m2 = jnp.max(probs2, axis=-1, keepdims=True)
    i2 = jnp.min(jnp.where(probs2 == m2, iota, big), axis=-1, keepdims=True)
    mask2 = iota == i2

    denom = m1 + m2
    wdense = jnp.where(mask1 | mask2, probs / denom, 0.0)
    for e in range(E_):
        wt_ref[e, :, 0] = wdense[:, e]


def _ffn_body(x_ref, w1_ref, w2_ref, b1_ref, b2_ref, wt_ref, out_ref):
    e = pl.program_id(0)
    hh = pl.program_id(1)
    is_first = jnp.logical_and(e == 0, hh == 0)

    @pl.when(is_first)
    def _():
        out_ref[...] = jnp.zeros_like(out_ref)

    wcol = wt_ref[0]
    h = jnp.dot(x_ref[...], w1_ref[0], preferred_element_type=jnp.float32)
    h = h + b1_ref[0]
    h = 0.5 * h * (1.0 + jax.lax.erf(h * (2.0 ** -0.5)))
    contrib = jnp.dot(h, w2_ref[0], preferred_element_type=jnp.float32)

    @pl.when(hh == 0)
    def _():
        out_ref[...] += wcol * b2_ref[0]

    out_ref[...] += wcol * contrib


@jax.jit
def kernel(x, gate_w, w1, w2, b1, b2):
    B, T, D = x.shape
    N = B * T
    x_flat = x.reshape(N, D)

    wt = pl.pallas_call(
        _router_body,
        out_shape=jax.ShapeDtypeStruct((E_, N, 1), jnp.float32),
    )(x_flat, gate_w)

    n_h = D_HID_ // H_TILE
    out = pl.pallas_call(
        _ffn_body,
        grid=(E_, n_h),
        in_specs=[
            pl.BlockSpec((N, D), lambda e, hh: (0, 0)),
            pl.BlockSpec((1, D, H_TILE), lambda e, hh: (e, 0, hh)),
            pl.BlockSpec((1, H_TILE, D), lambda e, hh: (e, hh, 0)),
            pl.BlockSpec((1, 1, H_TILE), lambda e, hh: (e, 0, hh)),
            pl.BlockSpec((1, 1, D), lambda e, hh: (e, 0, 0)),
            pl.BlockSpec((1, N, 1), lambda e, hh: (e, 0, 0)),
        ],
        out_specs=pl.BlockSpec((N, D), lambda e, hh: (0, 0)),
        out_shape=jax.ShapeDtypeStruct((N, D), jnp.float32),
        compiler_params=pltpu.CompilerParams(
            dimension_semantics=("arbitrary", "arbitrary")
        ),
    )(x_flat, w1, w2, b1, b2, wt)

    return out.reshape(B, T, D)
